# baseline (speedup 1.0000x reference)
import functools

import jax
import jax.numpy as jnp
from jax import lax
from jax.experimental import pallas as pl
from jax.experimental.pallas import tpu as pltpu
from jax.experimental.pallas import tpu_sc as plsc

N = 10000
E = 320000
H = 128
G = 64
DEPTH = 3
FFN_HIDDEN = 256

NB = 5
BLK = N // NB

NTILES = 32
EPT = E // NTILES
CH = 80
NCHP = 125
EPTP = NCHP * CH
NPAD = 10240
ROWS_PER_TILE = NPAD // 16


def _h0_body(x_ref, w_ref, b_ref, o_ref):
    o_ref[...] = jnp.maximum(
        jnp.dot(x_ref[...].astype(jnp.bfloat16),
                w_ref[...].astype(jnp.bfloat16),
                preferred_element_type=jnp.float32)
        + b_ref[...],
        0.0,
    )


@jax.jit
def _h0_call(x, W_in, b_in):
    return pl.pallas_call(
        _h0_body,
        grid=(NB,),
        in_specs=[
            pl.BlockSpec((BLK, H), lambda i: (i, 0)),
            pl.BlockSpec((H, H), lambda i: (0, 0)),
            pl.BlockSpec((1, H), lambda i: (0, 0)),
        ],
        out_specs=pl.BlockSpec((BLK, H), lambda i: (i, 0)),
        out_shape=jax.ShapeDtypeStruct((N, H), jnp.float32),
    )(x, W_in, b_in)


def _upd_body(h0_ref, p_ref, wh_ref, o_ref):
    agg = p_ref[0] + p_ref[1]
    o_ref[...] = jnp.maximum(
        h0_ref[...]
        + jnp.dot(agg.astype(jnp.bfloat16),
                  wh_ref[...].astype(jnp.bfloat16),
                  preferred_element_type=jnp.float32),
        0.0,
    )


@jax.jit
def _upd_call(h0, parts, W_h):
    return pl.pallas_call(
        _upd_body,
        grid=(NB,),
        in_specs=[
            pl.BlockSpec((BLK, H), lambda i: (i, 0)),
            pl.BlockSpec((2, BLK, H), lambda i: (0, i, 0)),
            pl.BlockSpec((H, H), lambda i: (0, 0)),
        ],
        out_specs=pl.BlockSpec((BLK, H), lambda i: (i, 0)),
        out_shape=jax.ShapeDtypeStruct((N, H), jnp.float32),
    )(h0, parts, W_h)


def _readout_body(h_ref, batch_ref, w1_ref, b1_ref, w2_ref, b2_ref, o_ref,
                  acc_ref, cnt_ref):
    i = pl.program_id(0)

    @pl.when(i == 0)
    def _():
        acc_ref[...] = jnp.zeros_like(acc_ref)
        cnt_ref[...] = jnp.zeros_like(cnt_ref)

    b = batch_ref[0, 0, :]
    seg = lax.broadcasted_iota(jnp.int32, (G, BLK), 0)
    mask = (seg == b[None, :]).astype(jnp.float32)
    acc_ref[...] += jnp.dot(mask, h_ref[...],
                            preferred_element_type=jnp.float32,
                precision=lax.Precision.HIGHEST)
    cnt_ref[...] += jnp.broadcast_to(
        jnp.sum(mask, axis=1, keepdims=True), (G, H))

    @pl.when(i == NB - 1)
    def _():
        cnt = jnp.maximum(cnt_ref[...], 1.0)
        mol = acc_ref[...] / cnt
        hid = jnp.maximum(
            jnp.dot(mol.astype(jnp.bfloat16),
                    w1_ref[...].astype(jnp.bfloat16),
                    preferred_element_type=jnp.float32)
            + b1_ref[...],
            0.0,
        )
        o_ref[...] = (
            jnp.dot(hid.astype(jnp.bfloat16),
                    w2_ref[...].astype(jnp.bfloat16),
                    preferred_element_type=jnp.float32)
            + b2_ref[...]
        )


@jax.jit
def _readout_call(h, batch3d, W1a, b1, W2, b2):
    return pl.pallas_call(
        _readout_body,
        grid=(NB,),
        in_specs=[
            pl.BlockSpec((BLK, H), lambda i: (i, 0)),
            pl.BlockSpec((1, 1, BLK), lambda i: (i, 0, 0)),
            pl.BlockSpec((H, FFN_HIDDEN), lambda i: (0, 0)),
            pl.BlockSpec((1, FFN_HIDDEN), lambda i: (0, 0)),
            pl.BlockSpec((FFN_HIDDEN, 1), lambda i: (0, 0)),
            pl.BlockSpec((1, 1), lambda i: (0, 0)),
        ],
        out_specs=pl.BlockSpec((G, 1), lambda i: (0, 0)),
        out_shape=jax.ShapeDtypeStruct((G, 1), jnp.float32),
        scratch_shapes=[
            pltpu.VMEM((G, H), jnp.float32),
            pltpu.VMEM((G, H), jnp.float32),
        ],
    )(h, batch3d, W1a, b1, W2, b2)


_sc_mesh = plsc.VectorSubcoreMesh(core_axis_name="c", subcore_axis_name="s")


@jax.jit
@functools.partial(
    pl.kernel,
    out_type=jax.ShapeDtypeStruct((2, NPAD, H), jnp.float32),
    mesh=_sc_mesh,
    scratch_types=[
        pltpu.VMEM((CH,), jnp.int32),
        pltpu.VMEM((CH,), jnp.int32),
        pltpu.VMEM((CH, H), jnp.float32),
        pltpu.VMEM_SHARED((NPAD, H), jnp.float32),
        pltpu.SemaphoreType.DMA,
    ],
)
def _sc_agg(h_hbm, src_hbm, dst_hbm, out_hbm,
            src_v, dst_v, rows_v, agg_sh, sem):
    c = lax.axis_index("c")
    s = lax.axis_index("s")
    wid = c * 16 + s

    def _zero_row(r, carry):
        for j in range(H // 16):
            rows_v[r, pl.ds(j * 16, 16)] = jnp.zeros((16,), jnp.float32)
        return carry

    lax.fori_loop(0, CH, _zero_row, 0)

    def _zero_stripe(k, carry):
        pltpu.sync_copy(
            rows_v, agg_sh.at[pl.ds(s * ROWS_PER_TILE + k * CH, CH)])
        return carry

    lax.fori_loop(0, ROWS_PER_TILE // CH, _zero_stripe, 0)
    plsc.subcore_barrier()

    base = wid * EPTP

    def _chunk(i, carry):
        off = base + i * CH
        pltpu.sync_copy(src_hbm.at[pl.ds(off, CH)], src_v)
        pltpu.sync_copy(dst_hbm.at[pl.ds(off, CH)], dst_v)
        pltpu.async_copy(h_hbm.at[src_v], rows_v, sem).wait()
        pltpu.sync_copy(rows_v, agg_sh.at[dst_v], add=True)
        return carry

    lax.fori_loop(0, NCHP, _chunk, 0)
    plsc.subcore_barrier()

    pltpu.sync_copy(
        agg_sh.at[pl.ds(s * ROWS_PER_TILE, ROWS_PER_TILE)],
        out_hbm.at[c, pl.ds(s * ROWS_PER_TILE, ROWS_PER_TILE)],
    )


def kernel(x, edge_index, batch, W_in, b_in, W_h, W1, b1, W2, b2):
    src = edge_index[0].astype(jnp.int32)
    dst = edge_index[1].astype(jnp.int32)
    pad_src = jnp.broadcast_to(
        jnp.arange(EPTP - EPT, dtype=jnp.int32), (NTILES, EPTP - EPT))
    src = jnp.concatenate(
        [src.reshape(NTILES, EPT), pad_src], axis=1).reshape(-1)
    pad_dst = jnp.broadcast_to(
        N + jnp.arange(EPTP - EPT, dtype=jnp.int32),
        (NTILES, EPTP - EPT))
    dst = jnp.concatenate(
        [dst.reshape(NTILES, EPT), pad_dst], axis=1).reshape(-1)
    batch3d = batch.astype(jnp.int32).reshape(NB, 1, BLK)

    h0 = _h0_call(x, W_in, b_in.reshape(1, H))
    h = h0
    for _ in range(DEPTH):
        parts = _sc_agg(h, src, dst)
        h = _upd_call(h0, parts, W_h)

    out = _readout_call(
        h,
        batch3d,
        W1[:H],
        b1.reshape(1, FFN_HIDDEN),
        W2,
        b2.reshape(1, 1),
    )
    return out


# speedup vs baseline: 2.2498x; 2.2498x over previous
import functools

import jax
import jax.numpy as jnp
from jax import lax
from jax.experimental import pallas as pl
from jax.experimental.pallas import tpu as pltpu
from jax.experimental.pallas import tpu_sc as plsc

N = 10000
E = 320000
H = 128
G = 64
DEPTH = 3
FFN_HIDDEN = 256

NB = 5
BLK = N // NB

NTILES = 32
EPT = E // NTILES
CH = 80
NCH_REAL = 126
NCH_TOT = NCH_REAL + 2
EPTP = NCH_REAL * CH
NPAD = 10240
ROWS_PER_TILE = NPAD // 16


def _h0_body(x_ref, w_ref, b_ref, o_ref):
    o_ref[...] = jnp.maximum(
        jnp.dot(x_ref[...].astype(jnp.bfloat16),
                w_ref[...].astype(jnp.bfloat16),
                preferred_element_type=jnp.float32)
        + b_ref[...],
        0.0,
    )


@jax.jit
def _h0_call(x, W_in, b_in):
    return pl.pallas_call(
        _h0_body,
        grid=(NB,),
        in_specs=[
            pl.BlockSpec((BLK, H), lambda i: (i, 0)),
            pl.BlockSpec((H, H), lambda i: (0, 0)),
            pl.BlockSpec((1, H), lambda i: (0, 0)),
        ],
        out_specs=pl.BlockSpec((BLK, H), lambda i: (i, 0)),
        out_shape=jax.ShapeDtypeStruct((N, H), jnp.float32),
    )(x, W_in, b_in)


def _upd_body(h0_ref, p_ref, wh_ref, o_ref):
    agg = p_ref[0] + p_ref[1]
    o_ref[...] = jnp.maximum(
        h0_ref[...]
        + jnp.dot(agg.astype(jnp.bfloat16),
                  wh_ref[...].astype(jnp.bfloat16),
                  preferred_element_type=jnp.float32),
        0.0,
    )


@jax.jit
def _upd_call(h0, parts, W_h):
    return pl.pallas_call(
        _upd_body,
        grid=(NB,),
        in_specs=[
            pl.BlockSpec((BLK, H), lambda i: (i, 0)),
            pl.BlockSpec((2, BLK, H), lambda i: (0, i, 0)),
            pl.BlockSpec((H, H), lambda i: (0, 0)),
        ],
        out_specs=pl.BlockSpec((BLK, H), lambda i: (i, 0)),
        out_shape=jax.ShapeDtypeStruct((N, H), jnp.float32),
    )(h0, parts, W_h)


def _readout_body(h_ref, batch_ref, w1_ref, b1_ref, w2_ref, b2_ref, o_ref,
                  acc_ref, cnt_ref):
    i = pl.program_id(0)

    @pl.when(i == 0)
    def _():
        acc_ref[...] = jnp.zeros_like(acc_ref)
        cnt_ref[...] = jnp.zeros_like(cnt_ref)

    b = batch_ref[0, 0, :]
    seg = lax.broadcasted_iota(jnp.int32, (G, BLK), 0)
    mask = (seg == b[None, :]).astype(jnp.float32)
    acc_ref[...] += jnp.dot(mask, h_ref[...],
                            preferred_element_type=jnp.float32,
                precision=lax.Precision.HIGHEST)
    cnt_ref[...] += jnp.broadcast_to(
        jnp.sum(mask, axis=1, keepdims=True), (G, H))

    @pl.when(i == NB - 1)
    def _():
        cnt = jnp.maximum(cnt_ref[...], 1.0)
        mol = acc_ref[...] / cnt
        hid = jnp.maximum(
            jnp.dot(mol.astype(jnp.bfloat16),
                    w1_ref[...].astype(jnp.bfloat16),
                    preferred_element_type=jnp.float32)
            + b1_ref[...],
            0.0,
        )
        o_ref[...] = (
            jnp.dot(hid.astype(jnp.bfloat16),
                    w2_ref[...].astype(jnp.bfloat16),
                    preferred_element_type=jnp.float32)
            + b2_ref[...]
        )


@jax.jit
def _readout_call(h, batch3d, W1a, b1, W2, b2):
    return pl.pallas_call(
        _readout_body,
        grid=(NB,),
        in_specs=[
            pl.BlockSpec((BLK, H), lambda i: (i, 0)),
            pl.BlockSpec((1, 1, BLK), lambda i: (i, 0, 0)),
            pl.BlockSpec((H, FFN_HIDDEN), lambda i: (0, 0)),
            pl.BlockSpec((1, FFN_HIDDEN), lambda i: (0, 0)),
            pl.BlockSpec((FFN_HIDDEN, 1), lambda i: (0, 0)),
            pl.BlockSpec((1, 1), lambda i: (0, 0)),
        ],
        out_specs=pl.BlockSpec((G, 1), lambda i: (0, 0)),
        out_shape=jax.ShapeDtypeStruct((G, 1), jnp.float32),
        scratch_shapes=[
            pltpu.VMEM((G, H), jnp.float32),
            pltpu.VMEM((G, H), jnp.float32),
        ],
    )(h, batch3d, W1a, b1, W2, b2)


_sc_mesh = plsc.VectorSubcoreMesh(core_axis_name="c", subcore_axis_name="s")


@jax.jit
@functools.partial(
    pl.kernel,
    out_type=jax.ShapeDtypeStruct((2, NPAD, H), jnp.float32),
    mesh=_sc_mesh,
    scratch_types=[
        pltpu.VMEM((CH,), jnp.int32),
        pltpu.VMEM((CH,), jnp.int32),
        pltpu.VMEM((NCH_REAL, CH), jnp.int32),
        pltpu.VMEM((CH, H), jnp.float32),
        pltpu.VMEM((CH, H), jnp.float32),
        pltpu.VMEM_SHARED((NPAD, H), jnp.float32),
        pltpu.SemaphoreType.DMA,
        pltpu.SemaphoreType.DMA,
        pltpu.SemaphoreType.DMA,
        pltpu.SemaphoreType.DMA,
    ],
)
def _sc_agg(h_hbm, src_hbm, dst_hbm, out_hbm,
            src_v0, src_v1, dst_i, rows0, rows1, agg_sh,
            sem0, sem1, semi0, semi1):
    c = lax.axis_index("c")
    s = lax.axis_index("s")
    wid = c * 16 + s

    def _zero_row(r, carry):
        for j in range(H // 16):
            rows0[r, pl.ds(j * 16, 16)] = jnp.zeros((16,), jnp.float32)
        return carry

    lax.fori_loop(0, CH, _zero_row, 0)

    def _zero_stripe(k, carry):
        pltpu.sync_copy(
            rows0, agg_sh.at[pl.ds(s * ROWS_PER_TILE + k * CH, CH)])
        return carry

    lax.fori_loop(0, ROWS_PER_TILE // CH, _zero_stripe, 0)

    pltpu.sync_copy(dst_hbm.at[wid], dst_i)

    base = wid * NCH_TOT * CH
    pltpu.sync_copy(src_hbm.at[pl.ds(base, CH)], src_v0)
    pltpu.sync_copy(src_hbm.at[pl.ds(base + CH, CH)], src_v1)
    pltpu.async_copy(h_hbm.at[src_v0], rows0, sem0)
    pltpu.async_copy(h_hbm.at[src_v1], rows1, sem1)

    plsc.subcore_barrier()

    def _pair(k, carry):
        c0 = 2 * k
        c1 = c0 + 1
        pltpu.make_async_copy(h_hbm.at[src_v0], rows0, sem0).wait()
        pltpu.async_copy(
            src_hbm.at[pl.ds(base + (c0 + 2) * CH, CH)], src_v0, semi0)
        pltpu.sync_copy(rows0, agg_sh.at[dst_i.at[c0]], add=True)
        pltpu.make_async_copy(
            src_hbm.at[pl.ds(base + (c0 + 2) * CH, CH)], src_v0,
            semi0).wait()
        pltpu.async_copy(h_hbm.at[src_v0], rows0, sem0)
        pltpu.make_async_copy(h_hbm.at[src_v1], rows1, sem1).wait()
        pltpu.async_copy(
            src_hbm.at[pl.ds(base + (c1 + 2) * CH, CH)], src_v1, semi1)
        pltpu.sync_copy(rows1, agg_sh.at[dst_i.at[c1]], add=True)
        pltpu.make_async_copy(
            src_hbm.at[pl.ds(base + (c1 + 2) * CH, CH)], src_v1,
            semi1).wait()
        pltpu.async_copy(h_hbm.at[src_v1], rows1, sem1)
        return carry

    lax.fori_loop(0, NCH_REAL // 2, _pair, 0)

    pltpu.make_async_copy(h_hbm.at[src_v0], rows0, sem0).wait()
    pltpu.make_async_copy(h_hbm.at[src_v1], rows1, sem1).wait()

    plsc.subcore_barrier()

    pltpu.sync_copy(
        agg_sh.at[pl.ds(s * ROWS_PER_TILE, ROWS_PER_TILE)],
        out_hbm.at[c, pl.ds(s * ROWS_PER_TILE, ROWS_PER_TILE)],
    )


def kernel(x, edge_index, batch, W_in, b_in, W_h, W1, b1, W2, b2):
    src = edge_index[0].astype(jnp.int32)
    dst = edge_index[1].astype(jnp.int32)
    pad_src = jnp.broadcast_to(
        jnp.arange(EPTP - EPT, dtype=jnp.int32), (NTILES, EPTP - EPT))
    junk_src = jnp.broadcast_to(
        jnp.arange(2 * CH, dtype=jnp.int32), (NTILES, 2 * CH))
    src = jnp.concatenate(
        [src.reshape(NTILES, EPT), pad_src, junk_src], axis=1).reshape(-1)
    pad_dst = jnp.broadcast_to(
        N + jnp.arange(EPTP - EPT, dtype=jnp.int32),
        (NTILES, EPTP - EPT))
    dst = jnp.concatenate(
        [dst.reshape(NTILES, EPT), pad_dst],
        axis=1).reshape(NTILES, NCH_REAL, CH)
    batch3d = batch.astype(jnp.int32).reshape(NB, 1, BLK)

    h0 = _h0_call(x, W_in, b_in.reshape(1, H))
    h = h0
    for _ in range(DEPTH):
        parts = _sc_agg(h, src, dst)
        h = _upd_call(h0, parts, W_h)

    out = _readout_call(
        h,
        batch3d,
        W1[:H],
        b1.reshape(1, FFN_HIDDEN),
        W2,
        b2.reshape(1, 1),
    )
    return out


# speedup vs baseline: 2.2845x; 1.0154x over previous
import functools

import jax
import jax.numpy as jnp
from jax import lax
from jax.experimental import pallas as pl
from jax.experimental.pallas import tpu as pltpu
from jax.experimental.pallas import tpu_sc as plsc

N = 10000
E = 320000
H = 128
G = 64
DEPTH = 3
FFN_HIDDEN = 256

NB = 5
BLK = N // NB

NTILES = 32
EPT = E // NTILES
CH = 80
NCH_REAL = 126
NCH_TOT = NCH_REAL + 2
EPTP = NCH_REAL * CH
NPAD = 10240
ROWS_PER_TILE = NPAD // 16


def _h0_body(x_ref, w_ref, b_ref, o_ref):
    o_ref[...] = jnp.maximum(
        jnp.dot(x_ref[...].astype(jnp.bfloat16),
                w_ref[...].astype(jnp.bfloat16),
                preferred_element_type=jnp.float32)
        + b_ref[...],
        0.0,
    )


@jax.jit
def _h0_call(x, W_in, b_in):
    return pl.pallas_call(
        _h0_body,
        grid=(NB,),
        in_specs=[
            pl.BlockSpec((BLK, H), lambda i: (i, 0)),
            pl.BlockSpec((H, H), lambda i: (0, 0)),
            pl.BlockSpec((1, H), lambda i: (0, 0)),
        ],
        out_specs=pl.BlockSpec((BLK, H), lambda i: (i, 0)),
        out_shape=jax.ShapeDtypeStruct((N, H), jnp.float32),
    )(x, W_in, b_in)


def _upd_body(h0_ref, p_ref, wh_ref, o_ref):
    agg = p_ref[0] + p_ref[1]
    o_ref[...] = jnp.maximum(
        h0_ref[...]
        + jnp.dot(agg.astype(jnp.bfloat16),
                  wh_ref[...].astype(jnp.bfloat16),
                  preferred_element_type=jnp.float32),
        0.0,
    )


@jax.jit
def _upd_call(h0, parts, W_h):
    return pl.pallas_call(
        _upd_body,
        grid=(NB,),
        in_specs=[
            pl.BlockSpec((BLK, H), lambda i: (i, 0)),
            pl.BlockSpec((2, BLK, H), lambda i: (0, i, 0)),
            pl.BlockSpec((H, H), lambda i: (0, 0)),
        ],
        out_specs=pl.BlockSpec((BLK, H), lambda i: (i, 0)),
        out_shape=jax.ShapeDtypeStruct((N, H), jnp.float32),
    )(h0, parts, W_h)


def _readout_body(h0_ref, p_ref, wh_ref, batch_ref, w1_ref, b1_ref,
                  w2_ref, b2_ref, o_ref, acc_ref, cnt_ref):
    i = pl.program_id(0)

    @pl.when(i == 0)
    def _():
        acc_ref[...] = jnp.zeros_like(acc_ref)
        cnt_ref[...] = jnp.zeros_like(cnt_ref)

    agg = p_ref[0] + p_ref[1]
    h = jnp.maximum(
        h0_ref[...]
        + jnp.dot(agg.astype(jnp.bfloat16),
                  wh_ref[...].astype(jnp.bfloat16),
                  preferred_element_type=jnp.float32),
        0.0,
    )

    b = batch_ref[0, 0, :]
    seg = lax.broadcasted_iota(jnp.int32, (G, BLK), 0)
    mask = (seg == b[None, :]).astype(jnp.float32)
    acc_ref[...] += jnp.dot(mask, h,
                            preferred_element_type=jnp.float32,
                precision=lax.Precision.HIGHEST)
    cnt_ref[...] += jnp.broadcast_to(
        jnp.sum(mask, axis=1, keepdims=True), (G, H))

    @pl.when(i == NB - 1)
    def _():
        cnt = jnp.maximum(cnt_ref[...], 1.0)
        mol = acc_ref[...] / cnt
        hid = jnp.maximum(
            jnp.dot(mol.astype(jnp.bfloat16),
                    w1_ref[...].astype(jnp.bfloat16),
                    preferred_element_type=jnp.float32)
            + b1_ref[...],
            0.0,
        )
        o_ref[...] = (
            jnp.dot(hid.astype(jnp.bfloat16),
                    w2_ref[...].astype(jnp.bfloat16),
                    preferred_element_type=jnp.float32)
            + b2_ref[...]
        )


@jax.jit
def _readout_call(h0, parts, W_h, batch3d, W1a, b1, W2, b2):
    return pl.pallas_call(
        _readout_body,
        grid=(NB,),
        in_specs=[
            pl.BlockSpec((BLK, H), lambda i: (i, 0)),
            pl.BlockSpec((2, BLK, H), lambda i: (0, i, 0)),
            pl.BlockSpec((H, H), lambda i: (0, 0)),
            pl.BlockSpec((1, 1, BLK), lambda i: (i, 0, 0)),
            pl.BlockSpec((H, FFN_HIDDEN), lambda i: (0, 0)),
            pl.BlockSpec((1, FFN_HIDDEN), lambda i: (0, 0)),
            pl.BlockSpec((FFN_HIDDEN, 1), lambda i: (0, 0)),
            pl.BlockSpec((1, 1), lambda i: (0, 0)),
        ],
        out_specs=pl.BlockSpec((G, 1), lambda i: (0, 0)),
        out_shape=jax.ShapeDtypeStruct((G, 1), jnp.float32),
        scratch_shapes=[
            pltpu.VMEM((G, H), jnp.float32),
            pltpu.VMEM((G, H), jnp.float32),
        ],
    )(h0, parts, W_h, batch3d, W1a, b1, W2, b2)


_sc_mesh = plsc.VectorSubcoreMesh(core_axis_name="c", subcore_axis_name="s")


@jax.jit
@functools.partial(
    pl.kernel,
    out_type=jax.ShapeDtypeStruct((2, NPAD, H), jnp.float32),
    mesh=_sc_mesh,
    scratch_types=[
        pltpu.VMEM((CH,), jnp.int32),
        pltpu.VMEM((CH,), jnp.int32),
        pltpu.VMEM((NCH_REAL, CH), jnp.int32),
        pltpu.VMEM((CH, H), jnp.float32),
        pltpu.VMEM((CH, H), jnp.float32),
        pltpu.VMEM_SHARED((NPAD, H), jnp.float32),
        pltpu.SemaphoreType.DMA,
        pltpu.SemaphoreType.DMA,
        pltpu.SemaphoreType.DMA,
        pltpu.SemaphoreType.DMA,
    ],
)
def _sc_agg(h_hbm, src_hbm, dst_hbm, out_hbm,
            src_v0, src_v1, dst_i, rows0, rows1, agg_sh,
            sem0, sem1, semi0, semi1):
    c = lax.axis_index("c")
    s = lax.axis_index("s")
    wid = c * 16 + s

    def _zero_row(r, carry):
        for j in range(H // 16):
            rows0[r, pl.ds(j * 16, 16)] = jnp.zeros((16,), jnp.float32)
        return carry

    lax.fori_loop(0, CH, _zero_row, 0)

    def _zero_stripe(k, carry):
        pltpu.sync_copy(
            rows0, agg_sh.at[pl.ds(s * ROWS_PER_TILE + k * CH, CH)])
        return carry

    lax.fori_loop(0, ROWS_PER_TILE // CH, _zero_stripe, 0)

    pltpu.sync_copy(dst_hbm.at[wid], dst_i)

    base = wid * NCH_TOT * CH
    pltpu.sync_copy(src_hbm.at[pl.ds(base, CH)], src_v0)
    pltpu.sync_copy(src_hbm.at[pl.ds(base + CH, CH)], src_v1)
    pltpu.async_copy(h_hbm.at[src_v0], rows0, sem0)
    pltpu.async_copy(h_hbm.at[src_v1], rows1, sem1)

    plsc.subcore_barrier()

    def _pair(k, carry):
        c0 = 2 * k
        c1 = c0 + 1
        pltpu.make_async_copy(h_hbm.at[src_v0], rows0, sem0).wait()
        pltpu.async_copy(
            src_hbm.at[pl.ds(base + (c0 + 2) * CH, CH)], src_v0, semi0)
        pltpu.sync_copy(rows0, agg_sh.at[dst_i.at[c0]], add=True)
        pltpu.make_async_copy(
            src_hbm.at[pl.ds(base + (c0 + 2) * CH, CH)], src_v0,
            semi0).wait()
        pltpu.async_copy(h_hbm.at[src_v0], rows0, sem0)
        pltpu.make_async_copy(h_hbm.at[src_v1], rows1, sem1).wait()
        pltpu.async_copy(
            src_hbm.at[pl.ds(base + (c1 + 2) * CH, CH)], src_v1, semi1)
        pltpu.sync_copy(rows1, agg_sh.at[dst_i.at[c1]], add=True)
        pltpu.make_async_copy(
            src_hbm.at[pl.ds(base + (c1 + 2) * CH, CH)], src_v1,
            semi1).wait()
        pltpu.async_copy(h_hbm.at[src_v1], rows1, sem1)
        return carry

    lax.fori_loop(0, NCH_REAL // 2, _pair, 0)

    pltpu.make_async_copy(h_hbm.at[src_v0], rows0, sem0).wait()
    pltpu.make_async_copy(h_hbm.at[src_v1], rows1, sem1).wait()

    plsc.subcore_barrier()

    pltpu.sync_copy(
        agg_sh.at[pl.ds(s * ROWS_PER_TILE, ROWS_PER_TILE)],
        out_hbm.at[c, pl.ds(s * ROWS_PER_TILE, ROWS_PER_TILE)],
    )


def kernel(x, edge_index, batch, W_in, b_in, W_h, W1, b1, W2, b2):
    src = edge_index[0].astype(jnp.int32)
    dst = edge_index[1].astype(jnp.int32)
    pad_src = jnp.broadcast_to(
        jnp.arange(EPTP - EPT, dtype=jnp.int32), (NTILES, EPTP - EPT))
    junk_src = jnp.broadcast_to(
        jnp.arange(2 * CH, dtype=jnp.int32), (NTILES, 2 * CH))
    src = jnp.concatenate(
        [src.reshape(NTILES, EPT), pad_src, junk_src], axis=1).reshape(-1)
    pad_dst = jnp.broadcast_to(
        N + jnp.arange(EPTP - EPT, dtype=jnp.int32),
        (NTILES, EPTP - EPT))
    dst = jnp.concatenate(
        [dst.reshape(NTILES, EPT), pad_dst],
        axis=1).reshape(NTILES, NCH_REAL, CH)
    batch3d = batch.astype(jnp.int32).reshape(NB, 1, BLK)

    h0 = _h0_call(x, W_in, b_in.reshape(1, H))
    h = h0
    for _ in range(DEPTH - 1):
        parts = _sc_agg(h, src, dst)
        h = _upd_call(h0, parts, W_h)
    parts = _sc_agg(h, src, dst)

    out = _readout_call(
        h0,
        parts,
        W_h,
        batch3d,
        W1[:H],
        b1.reshape(1, FFN_HIDDEN),
        W2,
        b2.reshape(1, 1),
    )
    return out
